# Initial kernel scaffold; baseline (speedup 1.0000x reference)
import jax
import jax.numpy as jnp
from jax.experimental import pallas as pl


def kernel(visual_features, importance_scores):
    raise NotImplementedError("write your pallas kernel here")


# speedup vs baseline: 3.2135x; 3.2135x over previous
import jax
import jax.numpy as jnp
from jax import lax
from jax.experimental import pallas as pl
from jax.experimental.pallas import tpu as pltpu
from jax.experimental.pallas import tpu_sc as plsc

B, N, D = 64, 8192, 128
K = 3276
KPAD = 3280
IPAD = 4096
NC, NS, L = 2, 16, 16
NW = NC * NS
ROWS_PER_W = B // NW
LANE_BLK = N // L
CHUNK = 512
NR = N // D


def _body(vf, scores, sel, mask_out, idx_out,
          srow, key_a, key_b, pay_a, pay_b, hist, start, gidx, istage,
          rowbuf, sem):
    wid = lax.axis_index("s") * NC + lax.axis_index("c")
    lane = lax.iota(jnp.int32, 16)
    pos0 = lane * LANE_BLK
    ones_i = jnp.ones((16,), jnp.int32)
    ones_f = jnp.ones((16,), jnp.float32)
    zeros_i = jnp.zeros((16,), jnp.int32)
    zeros_f = jnp.zeros((16,), jnp.float32)

    for r in range(ROWS_PER_W):
        b = wid + NW * r
        pltpu.sync_copy(scores.at[pl.ds(b * NR, NR)], srow)

        def build_keys(i, _):
            x = srow[i // 8, pl.ds((i % 8) * 16, 16)]
            bi = lax.bitcast_convert_type(x, jnp.int32)
            key_a[pl.ds(i * 16, 16)] = jnp.where(bi < 0, bi, bi ^ 0x7FFFFFFF)
            return 0
        lax.fori_loop(0, N // 16, build_keys, 0)

        bufs = [(key_a, pay_a, key_b, pay_b), (key_b, pay_b, key_a, pay_a)]
        for p in range(4):
            shift = 8 * p
            ksrc, psrc, kdst, pdst = bufs[p % 2]

            def zero_hist(i, _):
                hist[pl.ds(i * 16, 16)] = zeros_i
                return 0
            lax.fori_loop(0, 256, zero_hist, 0)

            def histogram(i, _):
                k = plsc.load_gather(ksrc, [pos0 + i])
                d = lax.shift_right_logical(k, shift) & 255
                plsc.addupdate_scatter(hist, [(d << 4) | lane], ones_i)
                return 0
            lax.fori_loop(0, LANE_BLK, histogram, 0)

            def prefix(dd, g):
                h = hist[pl.ds(dd * 16, 16)]
                cs = plsc.cumsum(h)
                start[pl.ds(dd * 16, 16)] = cs - h + g
                return g + jnp.sum(h)
            lax.fori_loop(0, 256, prefix, jnp.int32(0))

            def permute(i, _):
                posv = pos0 + i
                k = plsc.load_gather(ksrc, [posv])
                pv = posv if p == 0 else plsc.load_gather(psrc, [posv])
                d = lax.shift_right_logical(k, shift) & 255
                hidx = (d << 4) | lane
                off = plsc.load_gather(start, [hidx])
                plsc.store_scatter(kdst, [off], k)
                plsc.store_scatter(pdst, [off], pv)
                plsc.store_scatter(start, [hidx], off + ones_i)
                return 0
            lax.fori_loop(0, LANE_BLK, permute, 0)

        def stage_idx(i, _):
            istage[i // 8, pl.ds((i % 8) * 16, 16)] = pay_a[pl.ds(i * 16, 16)]
            return 0
        lax.fori_loop(0, IPAD // 16, stage_idx, 0)
        pltpu.sync_copy(istage, idx_out.at[pl.ds(b * (IPAD // D), IPAD // D)])

        def mask_zero(i, _):
            srow[i // 8, pl.ds((i % 8) * 16, 16)] = zeros_f
            return 0
        lax.fori_loop(0, N // 16, mask_zero, 0)

        def mask_scatter(i, _):
            pv = pay_a[pl.ds(i * 16, 16)]
            plsc.store_scatter(srow, [pv >> 7, pv & 127], ones_f,
                               mask=(i * 16 + lane) < K)
            return 0
        lax.fori_loop(0, KPAD // 16, mask_scatter, 0)
        pltpu.sync_copy(srow, mask_out.at[pl.ds(b * NR, NR)])

        boff = b * N
        def flat_idx(i, _):
            gidx[pl.ds(i * 16, 16)] = pay_a[pl.ds(i * 16, 16)] + boff
            return 0
        lax.fori_loop(0, KPAD // 16, flat_idx, 0)

        off = 0
        for gsz, ssz in ((CHUNK, CHUNK),) * 6 + ((208, 204),):
            cp = pltpu.async_copy(vf.at[gidx.at[pl.ds(off, gsz)]],
                                  rowbuf.at[pl.ds(0, gsz)], sem)
            cp.wait()
            pltpu.sync_copy(rowbuf.at[pl.ds(0, ssz)],
                            sel.at[b, pl.ds(off, ssz)])
            off += ssz


_sc_call = pl.kernel(
    _body,
    out_type=(
        jax.ShapeDtypeStruct((B, K, D), jnp.float32),
        jax.ShapeDtypeStruct((B * NR, D), jnp.float32),
        jax.ShapeDtypeStruct((B * (IPAD // D), D), jnp.int32),
    ),
    mesh=plsc.VectorSubcoreMesh(core_axis_name="c", subcore_axis_name="s",
                                num_cores=NC, num_subcores=NS),
    compiler_params=pltpu.CompilerParams(needs_layout_passes=False),
    scratch_types=[
        pltpu.VMEM((NR, D), jnp.float32),
        pltpu.VMEM((N,), jnp.int32),
        pltpu.VMEM((N,), jnp.int32),
        pltpu.VMEM((N,), jnp.int32),
        pltpu.VMEM((N,), jnp.int32),
        pltpu.VMEM((4096,), jnp.int32),
        pltpu.VMEM((4096,), jnp.int32),
        pltpu.VMEM((KPAD,), jnp.int32),
        pltpu.VMEM((IPAD // D, D), jnp.int32),
        pltpu.VMEM((CHUNK, D), jnp.float32),
        pltpu.SemaphoreType.DMA,
    ],
)


def kernel(visual_features, importance_scores):
    vf_flat = visual_features.reshape(B * N, D)
    scores = importance_scores.reshape(B * NR, D)
    sel, mask2d, idx_pad = _sc_call(vf_flat, scores)
    return (sel,
            mask2d.reshape(B, N, 1),
            idx_pad.reshape(B, IPAD)[:, :K])


# speedup vs baseline: 3.2246x; 1.0034x over previous
import jax
import jax.numpy as jnp
from jax import lax
from jax.experimental import pallas as pl
from jax.experimental.pallas import tpu as pltpu
from jax.experimental.pallas import tpu_sc as plsc

B, N, D = 64, 8192, 128
K = 3276
KPAD = 3280
IPAD = 4096
NC, NS, L = 2, 16, 16
NW = NC * NS
ROWS_PER_W = B // NW
LANE_BLK = N // L
CHUNK = 512
NR = N // D


def _body(vf, scores, sel, mask_out, idx_out,
          srow, key_a, key_b, pay_a, pay_b, hist, start, istage,
          rowbuf, sem):
    wid = lax.axis_index("s") * NC + lax.axis_index("c")
    lane = lax.iota(jnp.int32, 16)
    pos0 = lane * LANE_BLK
    ones_i = jnp.ones((16,), jnp.int32)
    ones_f = jnp.ones((16,), jnp.float32)
    zeros_i = jnp.zeros((16,), jnp.int32)
    zeros_f = jnp.zeros((16,), jnp.float32)

    for r in range(ROWS_PER_W):
        b = wid + NW * r
        pltpu.sync_copy(scores.at[pl.ds(b * NR, NR)], srow)

        def build_keys(i, _):
            x = srow[i // 8, pl.ds((i % 8) * 16, 16)]
            bi = lax.bitcast_convert_type(x, jnp.int32)
            key_a[pl.ds(i * 16, 16)] = jnp.where(bi < 0, bi, bi ^ 0x7FFFFFFF)
            return 0
        lax.fori_loop(0, N // 16, build_keys, 0)

        bufs = [(key_a, pay_a, key_b, pay_b), (key_b, pay_b, key_a, pay_a)]
        for p in range(4):
            shift = 8 * p
            ksrc, psrc, kdst, pdst = bufs[p % 2]

            def zero_hist(i, _):
                hist[pl.ds(i * 16, 16)] = zeros_i
                return 0
            lax.fori_loop(0, 256, zero_hist, 0)

            def histogram(i, _):
                k = plsc.load_gather(ksrc, [pos0 + i])
                d = lax.shift_right_logical(k, shift) & 255
                plsc.addupdate_scatter(hist, [(d << 4) | lane], ones_i)
                return 0
            lax.fori_loop(0, LANE_BLK, histogram, 0)

            def prefix(dd, g):
                h = hist[pl.ds(dd * 16, 16)]
                cs = plsc.cumsum(h)
                start[pl.ds(dd * 16, 16)] = cs - h + g
                return g + jnp.sum(h)
            lax.fori_loop(0, 256, prefix, jnp.int32(0))

            def permute(i, _):
                posv = pos0 + i
                k = plsc.load_gather(ksrc, [posv])
                pv = posv if p == 0 else plsc.load_gather(psrc, [posv])
                d = lax.shift_right_logical(k, shift) & 255
                hidx = (d << 4) | lane
                off = plsc.load_gather(start, [hidx])
                plsc.store_scatter(kdst, [off], k)
                plsc.store_scatter(pdst, [off], pv)
                plsc.store_scatter(start, [hidx], off + ones_i)
                return 0
            lax.fori_loop(0, LANE_BLK, permute, 0)

        def stage_idx(i, _):
            istage[i // 8, pl.ds((i % 8) * 16, 16)] = pay_a[pl.ds(i * 16, 16)]
            return 0
        lax.fori_loop(0, IPAD // 16, stage_idx, 0)
        pltpu.sync_copy(istage, idx_out.at[pl.ds(b * (IPAD // D), IPAD // D)])

        def mask_zero(i, _):
            srow[i // 8, pl.ds((i % 8) * 16, 16)] = zeros_f
            return 0
        lax.fori_loop(0, N // 16, mask_zero, 0)

        def mask_scatter(i, _):
            pv = pay_a[pl.ds(i * 16, 16)]
            plsc.store_scatter(srow, [pv >> 7, pv & 127], ones_f,
                               mask=(i * 16 + lane) < K)
            return 0
        lax.fori_loop(0, KPAD // 16, mask_scatter, 0)
        pltpu.sync_copy(srow, mask_out.at[pl.ds(b * NR, NR)])

        off = 0
        for gsz, ssz in ((CHUNK, CHUNK),) * 6 + ((208, 204),):
            cp = pltpu.async_copy(vf.at[b].at[pay_a.at[pl.ds(off, gsz)]],
                                  rowbuf.at[pl.ds(0, gsz)], sem)
            cp.wait()
            pltpu.sync_copy(rowbuf.at[pl.ds(0, ssz)],
                            sel.at[b, pl.ds(off, ssz)])
            off += ssz


_sc_call = pl.kernel(
    _body,
    out_type=(
        jax.ShapeDtypeStruct((B, K, D), jnp.float32),
        jax.ShapeDtypeStruct((B * NR, D), jnp.float32),
        jax.ShapeDtypeStruct((B * (IPAD // D), D), jnp.int32),
    ),
    mesh=plsc.VectorSubcoreMesh(core_axis_name="c", subcore_axis_name="s",
                                num_cores=NC, num_subcores=NS),
    compiler_params=pltpu.CompilerParams(needs_layout_passes=False),
    scratch_types=[
        pltpu.VMEM((NR, D), jnp.float32),
        pltpu.VMEM((N,), jnp.int32),
        pltpu.VMEM((N,), jnp.int32),
        pltpu.VMEM((N,), jnp.int32),
        pltpu.VMEM((N,), jnp.int32),
        pltpu.VMEM((4096,), jnp.int32),
        pltpu.VMEM((4096,), jnp.int32),
        pltpu.VMEM((IPAD // D, D), jnp.int32),
        pltpu.VMEM((CHUNK, D), jnp.float32),
        pltpu.SemaphoreType.DMA,
    ],
)


def kernel(visual_features, importance_scores):
    scores = importance_scores.reshape(B * NR, D)
    sel, mask2d, idx_pad = _sc_call(visual_features, scores)
    return (sel,
            mask2d.reshape(B, N, 1),
            idx_pad.reshape(B, IPAD)[:, :K])


# speedup vs baseline: 3.6954x; 1.1460x over previous
import jax
import jax.numpy as jnp
from jax import lax
from jax.experimental import pallas as pl
from jax.experimental.pallas import tpu as pltpu
from jax.experimental.pallas import tpu_sc as plsc

B, N, D = 64, 8192, 128
K = 3276
KPAD = 3280
IPAD = 4096
NC, NS, L = 2, 16, 16
NW = NC * NS
LANE_BLK = N // L
CHUNK = 128
NR = N // D


def _body(vf, scores, sel, mask_out, idx_out,
          srow, ka1, kb1, pa1, pb1, ka2, kb2, pa2, pb2,
          h1, s1, h2, s2, istage, rowbuf, gsem, osem):
    wid = lax.axis_index("s") * NC + lax.axis_index("c")
    lane = lax.iota(jnp.int32, 16)
    pos0 = lane * LANE_BLK
    ones_i = jnp.ones((16,), jnp.int32)
    ones_f = jnp.ones((16,), jnp.float32)
    zeros_i = jnp.zeros((16,), jnp.int32)
    zeros_f = jnp.zeros((16,), jnp.float32)
    b1 = wid
    b2 = wid + NW

    for b, ka in ((b1, ka1), (b2, ka2)):
        pltpu.sync_copy(scores.at[pl.ds(b * NR, NR)], srow)

        def build_keys(i, _, ka=ka):
            x = srow[i // 8, pl.ds((i % 8) * 16, 16)]
            bi = lax.bitcast_convert_type(x, jnp.int32)
            ka[pl.ds(i * 16, 16)] = jnp.where(bi < 0, bi, bi ^ 0x7FFFFFFF)
            return 0
        lax.fori_loop(0, N // 16, build_keys, 0)

    bufs = [((ka1, pa1, kb1, pb1), (ka2, pa2, kb2, pb2)),
            ((kb1, pb1, ka1, pa1), (kb2, pb2, ka2, pa2))]
    for p in range(4):
        shift = 8 * p
        (ks1, ps1, kd1, pd1), (ks2, ps2, kd2, pd2) = bufs[p % 2]

        def zero_hist(i, _):
            h1[pl.ds(i * 16, 16)] = zeros_i
            h2[pl.ds(i * 16, 16)] = zeros_i
            return 0
        lax.fori_loop(0, 256, zero_hist, 0)

        def histogram(i, _):
            posv = pos0 + i
            k1 = plsc.load_gather(ks1, [posv])
            k2 = plsc.load_gather(ks2, [posv])
            d1 = lax.shift_right_logical(k1, shift) & 255
            d2 = lax.shift_right_logical(k2, shift) & 255
            plsc.addupdate_scatter(h1, [(d1 << 4) | lane], ones_i)
            plsc.addupdate_scatter(h2, [(d2 << 4) | lane], ones_i)
            return 0
        lax.fori_loop(0, LANE_BLK, histogram, 0)

        def prefix(dd, g):
            g1, g2 = g
            a1 = h1[pl.ds(dd * 16, 16)]
            a2 = h2[pl.ds(dd * 16, 16)]
            c1 = plsc.cumsum(a1)
            c2 = plsc.cumsum(a2)
            s1[pl.ds(dd * 16, 16)] = c1 - a1 + g1
            s2[pl.ds(dd * 16, 16)] = c2 - a2 + g2
            return (g1 + jnp.sum(a1), g2 + jnp.sum(a2))
        lax.fori_loop(0, 256, prefix, (jnp.int32(0), jnp.int32(0)))

        def permute(i, _):
            posv = pos0 + i
            k1 = plsc.load_gather(ks1, [posv])
            k2 = plsc.load_gather(ks2, [posv])
            if p == 0:
                v1 = v2 = posv
            else:
                v1 = plsc.load_gather(ps1, [posv])
                v2 = plsc.load_gather(ps2, [posv])
            d1 = lax.shift_right_logical(k1, shift) & 255
            d2 = lax.shift_right_logical(k2, shift) & 255
            hx1 = (d1 << 4) | lane
            hx2 = (d2 << 4) | lane
            o1 = plsc.load_gather(s1, [hx1])
            o2 = plsc.load_gather(s2, [hx2])
            if p == 3:
                m1 = o1 < KPAD
                m2 = o2 < KPAD
                plsc.store_scatter(kd1, [o1], k1, mask=m1)
                plsc.store_scatter(kd2, [o2], k2, mask=m2)
                plsc.store_scatter(pd1, [o1], v1, mask=m1)
                plsc.store_scatter(pd2, [o2], v2, mask=m2)
            else:
                plsc.store_scatter(kd1, [o1], k1)
                plsc.store_scatter(kd2, [o2], k2)
                plsc.store_scatter(pd1, [o1], v1)
                plsc.store_scatter(pd2, [o2], v2)
            plsc.store_scatter(s1, [hx1], o1 + ones_i)
            plsc.store_scatter(s2, [hx2], o2 + ones_i)
            return 0
        lax.fori_loop(0, LANE_BLK, permute, 0)

    for b, pa in ((b1, pa1), (b2, pa2)):
        def stage_idx(i, _, pa=pa):
            istage[i // 8, pl.ds((i % 8) * 16, 16)] = pa[pl.ds(i * 16, 16)]
            return 0
        lax.fori_loop(0, IPAD // 16, stage_idx, 0)
        pltpu.sync_copy(istage, idx_out.at[pl.ds(b * (IPAD // D), IPAD // D)])

        def mask_zero(i, _):
            srow[i // 8, pl.ds((i % 8) * 16, 16)] = zeros_f
            return 0
        lax.fori_loop(0, N // 16, mask_zero, 0)

        def mask_scatter(i, _, pa=pa):
            pv = pa[pl.ds(i * 16, 16)]
            plsc.store_scatter(srow, [pv >> 7, pv & 127], ones_f,
                               mask=(i * 16 + lane) < K)
            return 0
        lax.fori_loop(0, KPAD // 16, mask_scatter, 0)
        pltpu.sync_copy(srow, mask_out.at[pl.ds(b * NR, NR)])

        chunks = [(c * CHUNK, CHUNK, CHUNK) for c in range(K // CHUNK)]
        chunks.append((K - K % CHUNK, 80, K % CHUNK))
        pend_g = []
        pend_o = [None, None]

        def issue(ci):
            off, gsz, _ = chunks[ci]
            h = ci % 2
            if pend_o[h] is not None:
                pend_o[h].wait()
                pend_o[h] = None
            pend_g.append(pltpu.async_copy(
                vf.at[b].at[pa.at[pl.ds(off, gsz)]],
                rowbuf.at[h, pl.ds(0, gsz)], gsem))

        issue(0)
        for ci in range(len(chunks)):
            if ci + 1 < len(chunks):
                issue(ci + 1)
            pend_g.pop(0).wait()
            off, _, ssz = chunks[ci]
            h = ci % 2
            pend_o[h] = pltpu.async_copy(rowbuf.at[h, pl.ds(0, ssz)],
                                         sel.at[b, pl.ds(off, ssz)], osem)
        for cp in pend_o:
            if cp is not None:
                cp.wait()


_sc_call = pl.kernel(
    _body,
    out_type=(
        jax.ShapeDtypeStruct((B, K, D), jnp.float32),
        jax.ShapeDtypeStruct((B * NR, D), jnp.float32),
        jax.ShapeDtypeStruct((B * (IPAD // D), D), jnp.int32),
    ),
    mesh=plsc.VectorSubcoreMesh(core_axis_name="c", subcore_axis_name="s",
                                num_cores=NC, num_subcores=NS),
    compiler_params=pltpu.CompilerParams(needs_layout_passes=False),
    scratch_types=[
        pltpu.VMEM((NR, D), jnp.float32),
        pltpu.VMEM((N,), jnp.int32),
        pltpu.VMEM((N,), jnp.int32),
        pltpu.VMEM((N,), jnp.int32),
        pltpu.VMEM((N,), jnp.int32),
        pltpu.VMEM((N,), jnp.int32),
        pltpu.VMEM((N,), jnp.int32),
        pltpu.VMEM((N,), jnp.int32),
        pltpu.VMEM((N,), jnp.int32),
        pltpu.VMEM((4096,), jnp.int32),
        pltpu.VMEM((4096,), jnp.int32),
        pltpu.VMEM((4096,), jnp.int32),
        pltpu.VMEM((4096,), jnp.int32),
        pltpu.VMEM((IPAD // D, D), jnp.int32),
        pltpu.VMEM((2, CHUNK, D), jnp.float32),
        pltpu.SemaphoreType.DMA,
        pltpu.SemaphoreType.DMA,
    ],
)


def kernel(visual_features, importance_scores):
    scores = importance_scores.reshape(B * NR, D)
    sel, mask2d, idx_pad = _sc_call(visual_features, scores)
    return (sel,
            mask2d.reshape(B, N, 1),
            idx_pad.reshape(B, IPAD)[:, :K])


# speedup vs baseline: 4.4842x; 1.2135x over previous
import jax
import jax.numpy as jnp
from jax import lax
from jax.experimental import pallas as pl
from jax.experimental.pallas import tpu as pltpu
from jax.experimental.pallas import tpu_sc as plsc

B, N, D = 64, 8192, 128
K = 3276
KPAD = 3280
IPAD = 4096
NC, NS, L = 2, 16, 16
NW = NC * NS
LANE_BLK = N // L
CHUNK = 96
NR = N // D


def _body(vf, scores, sel, mask_out, idx_out,
          srow, ka1, kb1, pa1, pb1, ka2, kb2, pa2, pb2,
          h1, s1, h2, s2, istage, rowbuf, oi0, oi1, oit, gsem, osem):
    wid = lax.axis_index("s") * NC + lax.axis_index("c")
    lane = lax.iota(jnp.int32, 16)
    pos0 = lane * LANE_BLK
    ones_i = jnp.ones((16,), jnp.int32)
    ones_f = jnp.ones((16,), jnp.float32)
    zeros_i = jnp.zeros((16,), jnp.int32)
    zeros_f = jnp.zeros((16,), jnp.float32)
    b1 = wid
    b2 = wid + NW

    for b, ka in ((b1, ka1), (b2, ka2)):
        pltpu.sync_copy(scores.at[pl.ds(b * NR, NR)], srow)

        def build_keys(i, _, ka=ka):
            x = srow[i // 8, pl.ds((i % 8) * 16, 16)]
            bi = lax.bitcast_convert_type(x, jnp.int32)
            ka[pl.ds(i * 16, 16)] = jnp.where(bi < 0, bi, bi ^ 0x7FFFFFFF)
            return 0
        lax.fori_loop(0, N // 16, build_keys, 0)

    bufs = [((ka1, pa1, kb1, pb1), (ka2, pa2, kb2, pb2)),
            ((kb1, pb1, ka1, pa1), (kb2, pb2, ka2, pa2))]
    for p in range(4):
        shift = 8 * p
        (ks1, ps1, kd1, pd1), (ks2, ps2, kd2, pd2) = bufs[p % 2]

        def zero_hist(i, _):
            h1[pl.ds(i * 16, 16)] = zeros_i
            h2[pl.ds(i * 16, 16)] = zeros_i
            return 0
        lax.fori_loop(0, 256, zero_hist, 0)

        def histogram(i, _):
            posv = pos0 + i
            k1 = plsc.load_gather(ks1, [posv])
            k2 = plsc.load_gather(ks2, [posv])
            d1 = lax.shift_right_logical(k1, shift) & 255
            d2 = lax.shift_right_logical(k2, shift) & 255
            plsc.addupdate_scatter(h1, [(d1 << 4) | lane], ones_i)
            plsc.addupdate_scatter(h2, [(d2 << 4) | lane], ones_i)
            return 0
        lax.fori_loop(0, LANE_BLK, histogram, 0)

        def prefix(dd, g):
            g1, g2 = g
            a1 = h1[pl.ds(dd * 16, 16)]
            a2 = h2[pl.ds(dd * 16, 16)]
            c1 = plsc.cumsum(a1)
            c2 = plsc.cumsum(a2)
            s1[pl.ds(dd * 16, 16)] = c1 - a1 + g1
            s2[pl.ds(dd * 16, 16)] = c2 - a2 + g2
            return (g1 + jnp.sum(a1), g2 + jnp.sum(a2))
        lax.fori_loop(0, 256, prefix, (jnp.int32(0), jnp.int32(0)))

        def permute(i, _):
            posv = pos0 + i
            k1 = plsc.load_gather(ks1, [posv])
            k2 = plsc.load_gather(ks2, [posv])
            if p == 0:
                v1 = v2 = posv
            else:
                v1 = plsc.load_gather(ps1, [posv])
                v2 = plsc.load_gather(ps2, [posv])
            d1 = lax.shift_right_logical(k1, shift) & 255
            d2 = lax.shift_right_logical(k2, shift) & 255
            hx1 = (d1 << 4) | lane
            hx2 = (d2 << 4) | lane
            o1 = plsc.load_gather(s1, [hx1])
            o2 = plsc.load_gather(s2, [hx2])
            if p == 3:
                m1 = o1 < KPAD
                m2 = o2 < KPAD
                plsc.store_scatter(kd1, [o1], k1, mask=m1)
                plsc.store_scatter(kd2, [o2], k2, mask=m2)
                plsc.store_scatter(pd1, [o1], v1, mask=m1)
                plsc.store_scatter(pd2, [o2], v2, mask=m2)
            else:
                plsc.store_scatter(kd1, [o1], k1)
                plsc.store_scatter(kd2, [o2], k2)
                plsc.store_scatter(pd1, [o1], v1)
                plsc.store_scatter(pd2, [o2], v2)
            plsc.store_scatter(s1, [hx1], o1 + ones_i)
            plsc.store_scatter(s2, [hx2], o2 + ones_i)
            return 0
        lax.fori_loop(0, LANE_BLK, permute, 0)

    for b, pa in ((b1, pa1), (b2, pa2)):
        def stage_idx(i, _, pa=pa):
            istage[i // 8, pl.ds((i % 8) * 16, 16)] = pa[pl.ds(i * 16, 16)]
            return 0
        lax.fori_loop(0, IPAD // 16, stage_idx, 0)
        pltpu.sync_copy(istage, idx_out.at[pl.ds(b * (IPAD // D), IPAD // D)])

        def mask_zero(i, _):
            srow[i // 8, pl.ds((i % 8) * 16, 16)] = zeros_f
            return 0
        lax.fori_loop(0, N // 16, mask_zero, 0)

        def mask_scatter(i, _, pa=pa):
            pv = pa[pl.ds(i * 16, 16)]
            plsc.store_scatter(srow, [pv >> 7, pv & 127], ones_f,
                               mask=(i * 16 + lane) < K)
            return 0
        lax.fori_loop(0, KPAD // 16, mask_scatter, 0)
        pltpu.sync_copy(srow, mask_out.at[pl.ds(b * NR, NR)])

        chunks = [(c * CHUNK, CHUNK, CHUNK) for c in range(K // CHUNK)]
        chunks.append((K - K % CHUNK, 16, K % CHUNK))
        pend_g = []
        pend_o = [None, None]

        def issue(ci):
            off, gsz, _ = chunks[ci]
            h = ci % 2
            if pend_o[h] is not None:
                pend_o[h].wait()
                pend_o[h] = None
            pend_g.append(pltpu.async_copy(
                vf.at[b].at[pa.at[pl.ds(off, gsz)]],
                rowbuf.at[h, pl.ds(0, gsz)], gsem))

        issue(0)
        for ci in range(len(chunks)):
            if ci + 1 < len(chunks):
                issue(ci + 1)
            pend_g.pop(0).wait()
            off, _, ssz = chunks[ci]
            h = ci % 2
            oi = oit if ssz != CHUNK else (oi0 if h == 0 else oi1)
            obase = (off + lane) * B + b

            def fill_oidx(i, _, oi=oi, obase=obase):
                oi[pl.ds(i * 16, 16)] = obase + i * (16 * B)
                return 0
            lax.fori_loop(0, ssz // 16, fill_oidx, 0)
            if ssz % 16:
                j = ssz // 16
                plsc.store_scatter(
                    oit, [j * 16 + lane], obase + j * (16 * B),
                    mask=lane < (ssz % 16))
            pend_o[h] = pltpu.async_copy(rowbuf.at[h, pl.ds(0, ssz)],
                                         sel.at[oi], osem)
        for cp in pend_o:
            if cp is not None:
                cp.wait()


_sc_call = pl.kernel(
    _body,
    out_type=(
        jax.ShapeDtypeStruct((K * B, D), jnp.float32),
        jax.ShapeDtypeStruct((B * NR, D), jnp.float32),
        jax.ShapeDtypeStruct((B * (IPAD // D), D), jnp.int32),
    ),
    mesh=plsc.VectorSubcoreMesh(core_axis_name="c", subcore_axis_name="s",
                                num_cores=NC, num_subcores=NS),
    compiler_params=pltpu.CompilerParams(needs_layout_passes=False),
    scratch_types=[
        pltpu.VMEM((NR, D), jnp.float32),
        pltpu.VMEM((N,), jnp.int32),
        pltpu.VMEM((N,), jnp.int32),
        pltpu.VMEM((N,), jnp.int32),
        pltpu.VMEM((N,), jnp.int32),
        pltpu.VMEM((N,), jnp.int32),
        pltpu.VMEM((N,), jnp.int32),
        pltpu.VMEM((N,), jnp.int32),
        pltpu.VMEM((N,), jnp.int32),
        pltpu.VMEM((4096,), jnp.int32),
        pltpu.VMEM((4096,), jnp.int32),
        pltpu.VMEM((4096,), jnp.int32),
        pltpu.VMEM((4096,), jnp.int32),
        pltpu.VMEM((IPAD // D, D), jnp.int32),
        pltpu.VMEM((2, CHUNK, D), jnp.float32),
        pltpu.VMEM((CHUNK,), jnp.int32),
        pltpu.VMEM((CHUNK,), jnp.int32),
        pltpu.VMEM((K % CHUNK,), jnp.int32),
        pltpu.SemaphoreType.DMA,
        pltpu.SemaphoreType.DMA,
    ],
)


def kernel(visual_features, importance_scores):
    scores = importance_scores.reshape(B * NR, D)
    selp, mask2d, idx_pad = _sc_call(visual_features, scores)
    return (selp.reshape(K, B, D).transpose(1, 0, 2),
            mask2d.reshape(B, N, 1),
            idx_pad.reshape(B, IPAD)[:, :K])


# speedup vs baseline: 5.4083x; 1.2061x over previous
import jax
import jax.numpy as jnp
from jax import lax
from jax.experimental import pallas as pl
from jax.experimental.pallas import tpu as pltpu
from jax.experimental.pallas import tpu_sc as plsc

B, N, D = 64, 8192, 128
K = 3276
KPAD = 3280
IPAD = 4096
NC, NS, L = 2, 16, 16
NW = NC * NS
LANE_BLK = N // L
HALF = LANE_BLK // 2
CHUNK = 96
NR = N // D
IMASK = (1 << 13) - 1


def _body(vf, scores, sel, mask_out, idx_out,
          srow, ka1, kb1, pb1, ka2, kb2, pb2,
          h1a, h1b, h2a, h2b, s1a, s1b, s2a, s2b,
          istage, gidx, rowbuf, oi0, oi1, oit, gsem, osem):
    wid = lax.axis_index("s") * NC + lax.axis_index("c")
    lane = lax.iota(jnp.int32, 16)
    posa0 = lane * LANE_BLK
    posb0 = posa0 + HALF
    ones_i = jnp.ones((16,), jnp.int32)
    ones_f = jnp.ones((16,), jnp.float32)
    zeros_i = jnp.zeros((16,), jnp.int32)
    zeros_f = jnp.zeros((16,), jnp.float32)
    b1 = wid
    b2 = wid + NW

    for b, ka in ((b1, ka1), (b2, ka2)):
        pltpu.sync_copy(scores.at[pl.ds(b * NR, NR)], srow)

        def build_keys(i, _, ka=ka):
            x = srow[i // 8, pl.ds((i % 8) * 16, 16)]
            bi = lax.bitcast_convert_type(x, jnp.int32)
            ka[pl.ds(i * 16, 16)] = jnp.where(bi < 0, bi, bi ^ 0x7FFFFFFF)
            return 0
        lax.fori_loop(0, N // 16, build_keys, 0)

    def zero_hist(i, _):
        for h in (h1a, h1b, h2a, h2b):
            h[pl.ds(i * 16, 16)] = zeros_i
        return 0
    lax.fori_loop(0, 256, zero_hist, 0)

    def dig0(w):
        return w & 255
    def dig1(w):
        return lax.shift_right_logical(w, 8) & 255
    def dig2(w):
        return lax.shift_right_logical(w, 13) & 255
    def dig3(w):
        return lax.shift_right_logical(w, 21) & 255
    passes = [
        (ka1, ka2, None, None, kb1, kb2, pb1, pb2, dig0, False),
        (kb1, kb2, pb1, pb2, ka1, ka2, None, None, dig1, True),
        (ka1, ka2, None, None, kb1, kb2, None, None, dig2, False),
        (kb1, kb2, None, None, ka1, ka2, None, None, dig3, False),
    ]
    for p, (ks1, ks2, ps1, ps2, kd1, kd2, pd1, pd2, dig, pack) \
            in enumerate(passes):

        def histogram(i, _):
            pv = []
            for u in range(2):
                pv += [posa0 + (i * 2 + u), posb0 + (i * 2 + u)]
            l1 = [plsc.load_gather(ks1, [q]) for q in pv]
            l2 = [plsc.load_gather(ks2, [q]) for q in pv]
            x1 = [(dig(k) << 4) | lane for k in l1]
            x2 = [(dig(k) << 4) | lane for k in l2]
            for u in range(2):
                plsc.addupdate_scatter(h1a, [x1[2 * u]], ones_i)
                plsc.addupdate_scatter(h1b, [x1[2 * u + 1]], ones_i)
                plsc.addupdate_scatter(h2a, [x2[2 * u]], ones_i)
                plsc.addupdate_scatter(h2b, [x2[2 * u + 1]], ones_i)
            return 0
        lax.fori_loop(0, HALF // 2, histogram, 0)

        def prefix(dd, g):
            g1, g2 = g
            sl = pl.ds(dd * 16, 16)
            a1 = h1a[sl]
            c1 = h1b[sl]
            a2 = h2a[sl]
            c2 = h2b[sl]
            t1 = a1 + c1
            t2 = a2 + c2
            e1 = plsc.cumsum(t1) - t1 + g1
            e2 = plsc.cumsum(t2) - t2 + g2
            s1a[sl] = e1
            s1b[sl] = e1 + a1
            s2a[sl] = e2
            s2b[sl] = e2 + a2
            h1a[sl] = zeros_i
            h1b[sl] = zeros_i
            h2a[sl] = zeros_i
            h2b[sl] = zeros_i
            return (g1 + jnp.sum(t1), g2 + jnp.sum(t2))
        lax.fori_loop(0, 256, prefix, (jnp.int32(0), jnp.int32(0)))

        def permute(i, _):
            pva = posa0 + i
            pvb = posb0 + i
            k1a = plsc.load_gather(ks1, [pva])
            k1b = plsc.load_gather(ks1, [pvb])
            k2a = plsc.load_gather(ks2, [pva])
            k2b = plsc.load_gather(ks2, [pvb])
            if ps1 is not None:
                v1a = plsc.load_gather(ps1, [pva])
                v1b = plsc.load_gather(ps1, [pvb])
                v2a = plsc.load_gather(ps2, [pva])
                v2b = plsc.load_gather(ps2, [pvb])
            x1a = (dig(k1a) << 4) | lane
            x1b = (dig(k1b) << 4) | lane
            x2a = (dig(k2a) << 4) | lane
            x2b = (dig(k2b) << 4) | lane
            o1a = plsc.load_gather(s1a, [x1a])
            o1b = plsc.load_gather(s1b, [x1b])
            o2a = plsc.load_gather(s2a, [x2a])
            o2b = plsc.load_gather(s2b, [x2b])
            if pack:
                w1a = (lax.shift_right_logical(k1a, 16) << 13) | v1a
                w1b = (lax.shift_right_logical(k1b, 16) << 13) | v1b
                w2a = (lax.shift_right_logical(k2a, 16) << 13) | v2a
                w2b = (lax.shift_right_logical(k2b, 16) << 13) | v2b
            else:
                w1a, w1b, w2a, w2b = k1a, k1b, k2a, k2b
            plsc.store_scatter(kd1, [o1a], w1a)
            plsc.store_scatter(kd1, [o1b], w1b)
            plsc.store_scatter(kd2, [o2a], w2a)
            plsc.store_scatter(kd2, [o2b], w2b)
            if pd1 is not None:
                plsc.store_scatter(pd1, [o1a], pva)
                plsc.store_scatter(pd1, [o1b], pvb)
                plsc.store_scatter(pd2, [o2a], pva)
                plsc.store_scatter(pd2, [o2b], pvb)
            plsc.store_scatter(s1a, [x1a], o1a + ones_i)
            plsc.store_scatter(s1b, [x1b], o1b + ones_i)
            plsc.store_scatter(s2a, [x2a], o2a + ones_i)
            plsc.store_scatter(s2b, [x2b], o2b + ones_i)
            return 0
        lax.fori_loop(0, HALF, permute, 0)

    for b, ka in ((b1, ka1), (b2, ka2)):
        def unpack(i, _, ka=ka):
            v = ka[pl.ds(i * 16, 16)] & IMASK
            gidx[pl.ds(i * 16, 16)] = v
            istage[i // 8, pl.ds((i % 8) * 16, 16)] = v
            return 0
        lax.fori_loop(0, IPAD // 16, unpack, 0)
        pltpu.sync_copy(istage, idx_out.at[pl.ds(b * (IPAD // D), IPAD // D)])

        def mask_zero(i, _):
            srow[i // 8, pl.ds((i % 8) * 16, 16)] = zeros_f
            return 0
        lax.fori_loop(0, N // 16, mask_zero, 0)

        def mask_scatter(i, _):
            pv = gidx[pl.ds(i * 16, 16)]
            plsc.store_scatter(srow, [pv >> 7, pv & 127], ones_f,
                               mask=(i * 16 + lane) < K)
            return 0
        lax.fori_loop(0, KPAD // 16, mask_scatter, 0)
        pltpu.sync_copy(srow, mask_out.at[pl.ds(b * NR, NR)])

        chunks = [(c * CHUNK, CHUNK, CHUNK) for c in range(K // CHUNK)]
        chunks.append((K - K % CHUNK, 16, K % CHUNK))
        pend_g = []
        pend_o = [None, None]

        def issue(ci):
            off, gsz, _ = chunks[ci]
            h = ci % 2
            if pend_o[h] is not None:
                pend_o[h].wait()
                pend_o[h] = None
            pend_g.append(pltpu.async_copy(
                vf.at[b].at[gidx.at[pl.ds(off, gsz)]],
                rowbuf.at[h, pl.ds(0, gsz)], gsem))

        issue(0)
        for ci in range(len(chunks)):
            if ci + 1 < len(chunks):
                issue(ci + 1)
            pend_g.pop(0).wait()
            off, _, ssz = chunks[ci]
            h = ci % 2
            oi = oit if ssz != CHUNK else (oi0 if h == 0 else oi1)
            obase = (off + lane) * B + b

            def fill_oidx(i, _, oi=oi, obase=obase):
                oi[pl.ds(i * 16, 16)] = obase + i * (16 * B)
                return 0
            lax.fori_loop(0, ssz // 16, fill_oidx, 0)
            if ssz % 16:
                j = ssz // 16
                plsc.store_scatter(
                    oit, [j * 16 + lane], obase + j * (16 * B),
                    mask=lane < (ssz % 16))
            pend_o[h] = pltpu.async_copy(rowbuf.at[h, pl.ds(0, ssz)],
                                         sel.at[oi], osem)
        for cp in pend_o:
            if cp is not None:
                cp.wait()


_sc_call = pl.kernel(
    _body,
    out_type=(
        jax.ShapeDtypeStruct((K * B, D), jnp.float32),
        jax.ShapeDtypeStruct((B * NR, D), jnp.float32),
        jax.ShapeDtypeStruct((B * (IPAD // D), D), jnp.int32),
    ),
    mesh=plsc.VectorSubcoreMesh(core_axis_name="c", subcore_axis_name="s",
                                num_cores=NC, num_subcores=NS),
    compiler_params=pltpu.CompilerParams(needs_layout_passes=False),
    scratch_types=[
        pltpu.VMEM((NR, D), jnp.float32),
        pltpu.VMEM((N,), jnp.int32),
        pltpu.VMEM((N,), jnp.int32),
        pltpu.VMEM((N,), jnp.int32),
        pltpu.VMEM((N,), jnp.int32),
        pltpu.VMEM((N,), jnp.int32),
        pltpu.VMEM((N,), jnp.int32),
        pltpu.VMEM((4096,), jnp.int32),
        pltpu.VMEM((4096,), jnp.int32),
        pltpu.VMEM((4096,), jnp.int32),
        pltpu.VMEM((4096,), jnp.int32),
        pltpu.VMEM((4096,), jnp.int32),
        pltpu.VMEM((4096,), jnp.int32),
        pltpu.VMEM((4096,), jnp.int32),
        pltpu.VMEM((4096,), jnp.int32),
        pltpu.VMEM((IPAD // D, D), jnp.int32),
        pltpu.VMEM((IPAD,), jnp.int32),
        pltpu.VMEM((2, CHUNK, D), jnp.float32),
        pltpu.VMEM((CHUNK,), jnp.int32),
        pltpu.VMEM((CHUNK,), jnp.int32),
        pltpu.VMEM((K % CHUNK,), jnp.int32),
        pltpu.SemaphoreType.DMA,
        pltpu.SemaphoreType.DMA,
    ],
)


def kernel(visual_features, importance_scores):
    scores = importance_scores.reshape(B * NR, D)
    selp, mask2d, idx_pad = _sc_call(visual_features, scores)
    return (selp.reshape(K, B, D).transpose(1, 0, 2),
            mask2d.reshape(B, N, 1),
            idx_pad.reshape(B, IPAD)[:, :K])
